# initial kernel scaffold (unmeasured)
import jax
import jax.numpy as jnp
from jax import lax
from jax.experimental import pallas as pl
from jax.experimental.pallas import tpu as pltpu

N_DEV = 8


def _ring_allgather(w_shard):
    e_loc, d, h_dim = w_shard.shape

    def body(w_ref, out_ref, comm_ref, send_sems, recv_sems, copy_sem, credit_sem):
        me = lax.axis_index("i")
        left = lax.rem(me + N_DEV - 1, N_DEV)
        right = lax.rem(me + 1, N_DEV)

        barrier_sem = pltpu.get_barrier_semaphore()
        for nbr in (left, right):
            pl.semaphore_signal(
                barrier_sem, inc=1,
                device_id=(nbr,), device_id_type=pl.DeviceIdType.MESH,
            )
        pl.semaphore_wait(barrier_sem, 2)

        comm_ref[0] = w_ref[...]
        own_cp = pltpu.make_async_copy(w_ref, out_ref.at[me], copy_sem)
        own_cp.start()
        own_cp.wait()

        for h in range(N_DEV - 1):
            s = h % 2
            r = (h + 1) % 2
            if h >= 1:
                pl.semaphore_wait(credit_sem, 1)
            rdma = pltpu.make_async_remote_copy(
                src_ref=comm_ref.at[s],
                dst_ref=comm_ref.at[r],
                send_sem=send_sems.at[h],
                recv_sem=recv_sems.at[h],
                device_id=(right,),
                device_id_type=pl.DeviceIdType.MESH,
            )
            rdma.start()
            rdma.wait()

            origin = lax.rem(me + N_DEV - h - 1, N_DEV)
            cp = pltpu.make_async_copy(comm_ref.at[r], out_ref.at[origin], copy_sem)
            cp.start()
            cp.wait()
            if h < N_DEV - 2:
                pl.semaphore_signal(
                    credit_sem, inc=1,
                    device_id=(left,), device_id_type=pl.DeviceIdType.MESH,
                )

    return pl.pallas_call(
        body,
        out_shape=jax.ShapeDtypeStruct((N_DEV, e_loc, d, h_dim), w_shard.dtype),
        in_specs=[pl.BlockSpec(memory_space=pltpu.VMEM)],
        out_specs=pl.BlockSpec(memory_space=pltpu.ANY),
        scratch_shapes=[
            pltpu.VMEM((2, e_loc, d, h_dim), w_shard.dtype),
            pltpu.SemaphoreType.DMA((N_DEV - 1,)),
            pltpu.SemaphoreType.DMA((N_DEV - 1,)),
            pltpu.SemaphoreType.DMA,
            pltpu.SemaphoreType.REGULAR,
        ],
        compiler_params=pltpu.CompilerParams(collective_id=0),
    )(w_shard)


def kernel(x, router_W, route_idx, expert_W, shared_W):
    n_tok, d_model = x.shape
    n_exp = router_W.shape[1]
    h_dim = shared_W.shape[1]

    w_full = _ring_allgather(expert_W).reshape(n_exp, d_model, h_dim)

    e = route_idx[:, 0]
    order = jnp.argsort(e)
    e_s = e[order]
    x_s = x[order]
    cap = 128
    pos = jnp.arange(n_tok, dtype=jnp.int32) - jnp.searchsorted(
        e_s, e_s, side="left"
    ).astype(jnp.int32)
    slot = e_s.astype(jnp.int32) * cap + jnp.minimum(pos, cap - 1)
    bins = jnp.zeros((n_exp * cap, d_model), x.dtype).at[slot].set(x_s)
    y_bins = jnp.einsum(
        "ecd,edh->ech",
        bins.reshape(n_exp, cap, d_model),
        w_full,
        preferred_element_type=jnp.float32,
    )
    y_s = y_bins.reshape(n_exp * cap, h_dim)[slot]
    y = jnp.zeros((n_tok, h_dim), x.dtype).at[order].set(y_s)

    probs = jax.nn.softmax(x @ router_W, axis=-1)
    p = jnp.take_along_axis(probs, route_idx, axis=1)
    return x @ shared_W + p * y


# baseline (device time: 897701 ns/iter reference)
import jax
import jax.numpy as jnp
from jax import lax
from jax.experimental import pallas as pl
from jax.experimental.pallas import tpu as pltpu

N_DEV = 8


def _ring_allgather(w_shard):
    e_loc, d, h_dim = w_shard.shape

    def body(w_ref, out_ref, comm_ref, send_sems, recv_sems, copy_sem, credit_sem):
        me = lax.axis_index("i")
        left = lax.rem(me + N_DEV - 1, N_DEV)
        right = lax.rem(me + 1, N_DEV)

        barrier_sem = pltpu.get_barrier_semaphore()
        for nbr in (left, right):
            pl.semaphore_signal(
                barrier_sem, inc=1,
                device_id=(nbr,), device_id_type=pl.DeviceIdType.MESH,
            )
        pl.semaphore_wait(barrier_sem, 2)

        comm_ref[0] = w_ref[...]
        own_cp = pltpu.make_async_copy(w_ref, out_ref.at[me], copy_sem)
        own_cp.start()
        own_cp.wait()

        for h in range(N_DEV - 1):
            s = h % 2
            r = (h + 1) % 2
            if h >= 1:
                pl.semaphore_wait(credit_sem, 1)
            rdma = pltpu.make_async_remote_copy(
                src_ref=comm_ref.at[s],
                dst_ref=comm_ref.at[r],
                send_sem=send_sems.at[h],
                recv_sem=recv_sems.at[h],
                device_id=(right,),
                device_id_type=pl.DeviceIdType.MESH,
            )
            rdma.start()
            rdma.wait()

            origin = lax.rem(me + N_DEV - h - 1, N_DEV)
            cp = pltpu.make_async_copy(comm_ref.at[r], out_ref.at[origin], copy_sem)
            cp.start()
            cp.wait()
            if h < N_DEV - 2:
                pl.semaphore_signal(
                    credit_sem, inc=1,
                    device_id=(left,), device_id_type=pl.DeviceIdType.MESH,
                )

    return pl.pallas_call(
        body,
        out_shape=jax.ShapeDtypeStruct((N_DEV, e_loc, d, h_dim), w_shard.dtype),
        in_specs=[pl.BlockSpec(memory_space=pltpu.VMEM)],
        out_specs=pl.BlockSpec(memory_space=pl.ANY),
        scratch_shapes=[
            pltpu.VMEM((2, e_loc, d, h_dim), w_shard.dtype),
            pltpu.SemaphoreType.DMA((N_DEV - 1,)),
            pltpu.SemaphoreType.DMA((N_DEV - 1,)),
            pltpu.SemaphoreType.DMA,
            pltpu.SemaphoreType.REGULAR,
        ],
        compiler_params=pltpu.CompilerParams(collective_id=0),
    )(w_shard)


def kernel(x, router_W, route_idx, expert_W, shared_W):
    n_tok, d_model = x.shape
    n_exp = router_W.shape[1]
    h_dim = shared_W.shape[1]

    w_full = _ring_allgather(expert_W).reshape(n_exp, d_model, h_dim)

    e = route_idx[:, 0]
    order = jnp.argsort(e)
    e_s = e[order]
    x_s = x[order]
    cap = 128
    pos = jnp.arange(n_tok, dtype=jnp.int32) - jnp.searchsorted(
        e_s, e_s, side="left"
    ).astype(jnp.int32)
    slot = e_s.astype(jnp.int32) * cap + jnp.minimum(pos, cap - 1)
    bins = jnp.zeros((n_exp * cap, d_model), x.dtype).at[slot].set(x_s)
    y_bins = jnp.einsum(
        "ecd,edh->ech",
        bins.reshape(n_exp, cap, d_model),
        w_full,
        preferred_element_type=jnp.float32,
    )
    y_s = y_bins.reshape(n_exp * cap, h_dim)[slot]
    y = jnp.zeros((n_tok, h_dim), x.dtype).at[order].set(y_s)

    probs = jax.nn.softmax(x @ router_W, axis=-1)
    p = jnp.take_along_axis(probs, route_idx, axis=1)
    return x @ shared_W + p * y


# device time: 578494 ns/iter; 1.5518x vs baseline; 1.5518x over previous
import jax
import jax.numpy as jnp
from jax import lax
from jax.experimental import pallas as pl
from jax.experimental.pallas import tpu as pltpu

N_DEV = 8
CAP = 96


def _moe_gemm_allgather(w_shard, bins):
    e_loc, d, h_dim = w_shard.shape
    n_exp = N_DEV * e_loc

    cw_hops = N_DEV // 2
    ccw_hops = N_DEV - 1 - cw_hops

    def body(w_ref, bins_ref, y_ref,
             cw_comm, ccw_comm,
             cw_send, cw_recv, ccw_send, ccw_recv,
             credit_cw, credit_ccw):
        me = lax.axis_index("i")
        left = lax.rem(me + N_DEV - 1, N_DEV)
        right = lax.rem(me + 1, N_DEV)

        def compute(origin, comm, slot):
            for e in range(e_loc):
                w_e = w_ref[e] if comm is None else comm[slot, e]
                row = (origin * e_loc + e) * CAP
                y_ref[pl.ds(row, CAP)] = jnp.dot(
                    bins_ref[pl.ds(row, CAP)], w_e,
                    preferred_element_type=jnp.float32,
                )

        barrier_sem = pltpu.get_barrier_semaphore()
        for nbr in (left, right):
            pl.semaphore_signal(
                barrier_sem, inc=1,
                device_id=(nbr,), device_id_type=pl.DeviceIdType.MESH,
            )
        pl.semaphore_wait(barrier_sem, 2)

        for r in range(cw_hops):
            if r >= 2:
                pl.semaphore_wait(credit_cw, 1)
            cw = pltpu.make_async_remote_copy(
                src_ref=cw_comm.at[(r - 1) % 2] if r else w_ref,
                dst_ref=cw_comm.at[r % 2],
                send_sem=cw_send.at[r],
                recv_sem=cw_recv.at[r],
                device_id=(right,),
                device_id_type=pl.DeviceIdType.MESH,
            )
            cw.start()
            ccw = None
            if r < ccw_hops:
                if r >= 2:
                    pl.semaphore_wait(credit_ccw, 1)
                ccw = pltpu.make_async_remote_copy(
                    src_ref=ccw_comm.at[(r - 1) % 2] if r else w_ref,
                    dst_ref=ccw_comm.at[r % 2],
                    send_sem=ccw_send.at[r],
                    recv_sem=ccw_recv.at[r],
                    device_id=(left,),
                    device_id_type=pl.DeviceIdType.MESH,
                )
                ccw.start()

            if r == 0:
                compute(me, None, 0)

            cw.wait()
            compute(lax.rem(me + N_DEV - r - 1, N_DEV), cw_comm, r % 2)
            if ccw is not None:
                ccw.wait()
                compute(lax.rem(me + r + 1, N_DEV), ccw_comm, r % 2)

            if 1 <= r < cw_hops - 1:
                pl.semaphore_signal(
                    credit_cw, inc=1,
                    device_id=(left,), device_id_type=pl.DeviceIdType.MESH,
                )
            if 1 <= r < ccw_hops - 1:
                pl.semaphore_signal(
                    credit_ccw, inc=1,
                    device_id=(right,), device_id_type=pl.DeviceIdType.MESH,
                )

    return pl.pallas_call(
        body,
        out_shape=jax.ShapeDtypeStruct((n_exp * CAP, h_dim), jnp.float32),
        in_specs=[
            pl.BlockSpec(memory_space=pltpu.VMEM),
            pl.BlockSpec(memory_space=pltpu.VMEM),
        ],
        out_specs=pl.BlockSpec(memory_space=pltpu.VMEM),
        scratch_shapes=[
            pltpu.VMEM((2, e_loc, d, h_dim), w_shard.dtype),
            pltpu.VMEM((2, e_loc, d, h_dim), w_shard.dtype),
            pltpu.SemaphoreType.DMA((cw_hops,)),
            pltpu.SemaphoreType.DMA((cw_hops,)),
            pltpu.SemaphoreType.DMA((ccw_hops,)),
            pltpu.SemaphoreType.DMA((ccw_hops,)),
            pltpu.SemaphoreType.REGULAR,
            pltpu.SemaphoreType.REGULAR,
        ],
        compiler_params=pltpu.CompilerParams(
            collective_id=0,
            vmem_limit_bytes=60 * 1024 * 1024,
        ),
    )(w_shard, bins)


def kernel(x, router_W, route_idx, expert_W, shared_W):
    n_tok, d_model = x.shape
    n_exp = router_W.shape[1]
    h_dim = shared_W.shape[1]

    e = route_idx[:, 0]
    order = jnp.argsort(e)
    e_s = e[order]
    x_s = x[order]
    pos = jnp.arange(n_tok, dtype=jnp.int32) - jnp.searchsorted(
        e_s, e_s, side="left"
    ).astype(jnp.int32)
    slot = e_s.astype(jnp.int32) * CAP + jnp.minimum(pos, CAP - 1)
    bins = jnp.zeros((n_exp * CAP, d_model), x.dtype).at[slot].set(x_s)

    y_bins = _moe_gemm_allgather(expert_W, bins)

    y_s = y_bins[slot]
    y = jnp.zeros((n_tok, h_dim), x.dtype).at[order].set(y_s)

    probs = jax.nn.softmax(x @ router_W, axis=-1)
    p = jnp.take_along_axis(probs, route_idx, axis=1)
    return x @ shared_W + p * y


# device time: 443039 ns/iter; 2.0262x vs baseline; 1.3057x over previous
import jax
import jax.numpy as jnp
from jax import lax
from jax.experimental import pallas as pl
from jax.experimental.pallas import tpu as pltpu

N_DEV = 8
CAP = 96


def _moe_gemm_allgather(w_shard, bins):
    e_loc, d, h_dim = w_shard.shape
    n_exp = N_DEV * e_loc

    cw_hops = N_DEV // 2
    ccw_hops = N_DEV - 1 - cw_hops

    def body(w_ref, bins_ref, y_ref,
             cw_comm, ccw_comm,
             cw_send, cw_recv, ccw_send, ccw_recv,
             credit_cw, credit_ccw):
        me = lax.axis_index("i")
        left = lax.rem(me + N_DEV - 1, N_DEV)
        right = lax.rem(me + 1, N_DEV)

        def compute(origin, comm, slot):
            for e in range(e_loc):
                w_e = w_ref[e] if comm is None else comm[slot, e]
                row = (origin * e_loc + e) * CAP
                y_ref[pl.ds(row, CAP)] = jnp.dot(
                    bins_ref[pl.ds(row, CAP)], w_e,
                    preferred_element_type=jnp.float32,
                )

        barrier_sem = pltpu.get_barrier_semaphore()
        for nbr in (left, right):
            pl.semaphore_signal(
                barrier_sem, inc=1,
                device_id=(nbr,), device_id_type=pl.DeviceIdType.MESH,
            )
        pl.semaphore_wait(barrier_sem, 2)

        for r in range(cw_hops):
            if r >= 2:
                pl.semaphore_wait(credit_cw, 1)
            cw = pltpu.make_async_remote_copy(
                src_ref=cw_comm.at[(r - 1) % 2] if r else w_ref,
                dst_ref=cw_comm.at[r % 2],
                send_sem=cw_send.at[r],
                recv_sem=cw_recv.at[r],
                device_id=(right,),
                device_id_type=pl.DeviceIdType.MESH,
            )
            cw.start()
            ccw = None
            if r < ccw_hops:
                if r >= 2:
                    pl.semaphore_wait(credit_ccw, 1)
                ccw = pltpu.make_async_remote_copy(
                    src_ref=ccw_comm.at[(r - 1) % 2] if r else w_ref,
                    dst_ref=ccw_comm.at[r % 2],
                    send_sem=ccw_send.at[r],
                    recv_sem=ccw_recv.at[r],
                    device_id=(left,),
                    device_id_type=pl.DeviceIdType.MESH,
                )
                ccw.start()

            if r == 0:
                compute(me, None, 0)

            cw.wait()
            compute(lax.rem(me + N_DEV - r - 1, N_DEV), cw_comm, r % 2)
            if ccw is not None:
                ccw.wait()
                compute(lax.rem(me + r + 1, N_DEV), ccw_comm, r % 2)

            if 1 <= r < cw_hops - 1:
                pl.semaphore_signal(
                    credit_cw, inc=1,
                    device_id=(left,), device_id_type=pl.DeviceIdType.MESH,
                )
            if 1 <= r < ccw_hops - 1:
                pl.semaphore_signal(
                    credit_ccw, inc=1,
                    device_id=(right,), device_id_type=pl.DeviceIdType.MESH,
                )

    return pl.pallas_call(
        body,
        out_shape=jax.ShapeDtypeStruct((n_exp * CAP, h_dim), jnp.float32),
        in_specs=[
            pl.BlockSpec(memory_space=pltpu.VMEM),
            pl.BlockSpec(memory_space=pltpu.VMEM),
        ],
        out_specs=pl.BlockSpec(memory_space=pltpu.VMEM),
        scratch_shapes=[
            pltpu.VMEM((2, e_loc, d, h_dim), w_shard.dtype),
            pltpu.VMEM((2, e_loc, d, h_dim), w_shard.dtype),
            pltpu.SemaphoreType.DMA((cw_hops,)),
            pltpu.SemaphoreType.DMA((cw_hops,)),
            pltpu.SemaphoreType.DMA((ccw_hops,)),
            pltpu.SemaphoreType.DMA((ccw_hops,)),
            pltpu.SemaphoreType.REGULAR,
            pltpu.SemaphoreType.REGULAR,
        ],
        compiler_params=pltpu.CompilerParams(
            collective_id=0,
            vmem_limit_bytes=60 * 1024 * 1024,
        ),
    )(w_shard, bins)


def kernel(x, router_W, route_idx, expert_W, shared_W):
    n_tok, d_model = x.shape
    n_exp = router_W.shape[1]
    h_dim = shared_W.shape[1]

    e = route_idx[:, 0].astype(jnp.int32)
    one_hot_e = (e[:, None] == jnp.arange(n_exp, dtype=jnp.int32)[None, :])
    pos = (
        jnp.take_along_axis(
            jnp.cumsum(one_hot_e.astype(jnp.int32), axis=0), e[:, None], axis=1
        )[:, 0]
        - 1
    )
    slot = e * CAP + jnp.minimum(pos, CAP - 1)
    disp = (
        slot[:, None] == jnp.arange(n_exp * CAP, dtype=jnp.int32)[None, :]
    ).astype(x.dtype)
    bins = disp.T @ x

    y_bins = _moe_gemm_allgather(expert_W, bins)

    y = disp @ y_bins

    probs = jax.nn.softmax(x @ router_W, axis=-1)
    p = jnp.take_along_axis(probs, route_idx, axis=1)
    return x @ shared_W + p * y


# device time: 157906 ns/iter; 5.6850x vs baseline; 2.8057x over previous
import jax
import jax.numpy as jnp
from jax import lax
from jax.experimental import pallas as pl
from jax.experimental.pallas import tpu as pltpu

N_DEV = 8
E_LOC = 4
CAP = 64


def _moe_a2a(bins, w_shard):
    _, _, _, d = bins.shape
    e_loc, _, h_dim = w_shard.shape

    def body(bins_ref, w_ref, out_ref, r_ref, y_ref,
             send_sems, recv_sems, back_send, back_recv):
        me = lax.axis_index("i")

        barrier_sem = pltpu.get_barrier_semaphore()
        for delta in range(1, N_DEV):
            pl.semaphore_signal(
                barrier_sem, inc=1,
                device_id=(lax.rem(me + delta, N_DEV),),
                device_id_type=pl.DeviceIdType.MESH,
            )
        pl.semaphore_wait(barrier_sem, N_DEV - 1)

        r_ref[pl.ds(me, 1)] = bins_ref[pl.ds(me, 1)]
        sends = []
        for delta in range(1, N_DEV):
            t = lax.rem(me + delta, N_DEV)
            rdma = pltpu.make_async_remote_copy(
                src_ref=bins_ref.at[t],
                dst_ref=r_ref.at[me],
                send_sem=send_sems.at[t],
                recv_sem=recv_sems.at[me],
                device_id=(t,),
                device_id_type=pl.DeviceIdType.MESH,
            )
            rdma.start()
            sends.append(rdma)
        for delta in range(1, N_DEV):
            s = lax.rem(me + N_DEV - delta, N_DEV)
            recv = pltpu.make_async_remote_copy(
                src_ref=bins_ref.at[s],
                dst_ref=r_ref.at[s],
                send_sem=send_sems.at[s],
                recv_sem=recv_sems.at[s],
                device_id=(s,),
                device_id_type=pl.DeviceIdType.MESH,
            )
            recv.wait_recv()

        for e in range(e_loc):
            x_e = r_ref[:, e].reshape(N_DEV * CAP, d)
            y_e = jnp.dot(x_e, w_ref[e], preferred_element_type=jnp.float32)
            y_ref[:, e] = y_e.reshape(N_DEV, CAP, h_dim)

        out_ref[pl.ds(me, 1)] = y_ref[pl.ds(me, 1)]
        for delta in range(1, N_DEV):
            t = lax.rem(me + delta, N_DEV)
            rdma = pltpu.make_async_remote_copy(
                src_ref=y_ref.at[t],
                dst_ref=out_ref.at[me],
                send_sem=back_send.at[t],
                recv_sem=back_recv.at[me],
                device_id=(t,),
                device_id_type=pl.DeviceIdType.MESH,
            )
            rdma.start()
            sends.append(rdma)
        for delta in range(1, N_DEV):
            s = lax.rem(me + N_DEV - delta, N_DEV)
            recv = pltpu.make_async_remote_copy(
                src_ref=y_ref.at[s],
                dst_ref=out_ref.at[s],
                send_sem=back_send.at[s],
                recv_sem=back_recv.at[s],
                device_id=(s,),
                device_id_type=pl.DeviceIdType.MESH,
            )
            recv.wait_recv()

        for rdma in sends:
            rdma.wait_send()

    return pl.pallas_call(
        body,
        out_shape=jax.ShapeDtypeStruct((N_DEV, e_loc, CAP, h_dim), jnp.float32),
        in_specs=[
            pl.BlockSpec(memory_space=pltpu.VMEM),
            pl.BlockSpec(memory_space=pltpu.VMEM),
        ],
        out_specs=pl.BlockSpec(memory_space=pltpu.VMEM),
        scratch_shapes=[
            pltpu.VMEM((N_DEV, e_loc, CAP, d), jnp.float32),
            pltpu.VMEM((N_DEV, e_loc, CAP, h_dim), jnp.float32),
            pltpu.SemaphoreType.DMA((N_DEV,)),
            pltpu.SemaphoreType.DMA((N_DEV,)),
            pltpu.SemaphoreType.DMA((N_DEV,)),
            pltpu.SemaphoreType.DMA((N_DEV,)),
        ],
        compiler_params=pltpu.CompilerParams(
            collective_id=0,
            vmem_limit_bytes=60 * 1024 * 1024,
        ),
    )(bins, w_shard)


def kernel(x, router_W, route_idx, expert_W, shared_W):
    n_tok, d_model = x.shape
    n_exp = router_W.shape[1]
    h_dim = shared_W.shape[1]

    e = route_idx[:, 0].astype(jnp.int32)
    one_hot_e = (e[:, None] == jnp.arange(n_exp, dtype=jnp.int32)[None, :])
    pos = (
        jnp.take_along_axis(
            jnp.cumsum(one_hot_e.astype(jnp.int32), axis=0), e[:, None], axis=1
        )[:, 0]
        - 1
    )
    slot = e * CAP + jnp.minimum(pos, CAP - 1)
    disp = (
        slot[:, None] == jnp.arange(n_exp * CAP, dtype=jnp.int32)[None, :]
    ).astype(x.dtype)
    bins = (disp.T @ x).reshape(N_DEV, E_LOC, CAP, d_model)

    y_slots = _moe_a2a(bins, expert_W)

    y = disp @ y_slots.reshape(n_exp * CAP, h_dim)

    probs = jax.nn.softmax(x @ router_W, axis=-1)
    p = jnp.take_along_axis(probs, route_idx, axis=1)
    return x @ shared_W + p * y
